# Initial kernel scaffold; baseline (speedup 1.0000x reference)
import jax
import jax.numpy as jnp
from jax.experimental import pallas as pl


def kernel(x, W):
    raise NotImplementedError("write your pallas kernel here")


# speedup vs baseline: 24.1783x; 24.1783x over previous
import jax
import jax.numpy as jnp
from jax.experimental import pallas as pl

_DIM = 32768
_C = 4
_BS = 8
_NSTEPS = 10
_INV_TEMP = 0.5
_INV_STEP = 5.0


def _pick4(planes, idx):
    return jnp.where(
        idx == 0, planes[0],
        jnp.where(idx == 1, planes[1],
                  jnp.where(idx == 2, planes[2], planes[3])))


def _log_softmax4(logits):
    m = jnp.maximum(jnp.maximum(logits[0], logits[1]),
                    jnp.maximum(logits[2], logits[3]))
    sh = [l - m for l in logits]
    se = ((jnp.exp(sh[0]) + jnp.exp(sh[1])) + jnp.exp(sh[2])) + jnp.exp(sh[3])
    lse = jnp.log(se)
    return [s - lse for s in sh]


def _step_kernel(gum_ref, u_ref, wp_ref, x_ref, out_ref):
    i = pl.program_id(0)

    @pl.when(i == 0)
    def _():
        out_ref[...] = x_ref[...]

    xc = out_ref[...]
    xc0 = xc[0:1, :]
    row0 = jax.lax.broadcasted_iota(jnp.int32, (_BS, 1), 0) == 0

    W_c = [wp_ref[c:c + 1, :] for c in range(_C)]
    G_c = [w * _INV_TEMP for w in W_c]

    Gc0 = _pick4(G_c, xc0)
    first = [g - Gc0 for g in G_c]
    lo_oth = [f - _INV_STEP for f in first]
    logits = [jnp.where(row0 & (xc0 == c), first[c], lo_oth[c])
              for c in range(_C)]

    gum = [gum_ref[0, c] for c in range(_C)]
    t0 = logits[0] + gum[0]
    xd = jnp.zeros((_BS, _DIM), jnp.int32)
    best = t0
    for c in range(1, _C):
        tc = logits[c] + gum[c]
        upd = tc > best
        xd = jnp.where(upd, c, xd)
        best = jnp.where(upd, tc, best)

    logp = _log_softmax4(logits)
    lp_fwd = jnp.sum(_pick4(logp, xd), axis=1, keepdims=True)

    xd0 = xd[0:1, :]
    Gd0 = _pick4(G_c, xd0)
    first_d = [g - Gd0 for g in G_c]
    lod_oth = [f - _INV_STEP for f in first_d]
    logits_d = [jnp.where(row0 & (xd0 == c), first_d[c], lod_oth[c])
                for c in range(_C)]
    logp_d = _log_softmax4(logits_d)
    lp_rev = jnp.sum(_pick4(logp_d, xc), axis=1, keepdims=True)

    e_d = jnp.sum(_pick4(W_c, xd0), axis=1, keepdims=True)
    e_c = jnp.sum(_pick4(W_c, xc0), axis=1, keepdims=True)
    m_term = e_d - e_c

    la = (m_term + lp_rev) - lp_fwd
    acc = jnp.exp(la) > u_ref[0]
    out_ref[...] = jnp.where(acc, xd, xc)


def kernel(x, W):
    xdtype = x.dtype
    xi = x.astype(jnp.int32)

    key = jax.random.key(42)
    gums, us = [], []
    for _ in range(_NSTEPS):
        key, ks, kr = jax.random.split(key, 3)
        gums.append(jax.random.gumbel(ks, (_BS, _DIM, _C), jnp.float32))
        us.append(jax.random.uniform(kr, (_BS,)))
    gum = jnp.stack(gums).transpose(0, 3, 1, 2)
    u = jnp.stack(us).reshape(_NSTEPS, _BS, 1)
    wp = jnp.concatenate([W.T, jnp.zeros((4, _DIM), jnp.float32)], axis=0)

    out = pl.pallas_call(
        _step_kernel,
        grid=(_NSTEPS,),
        in_specs=[
            pl.BlockSpec((1, _C, _BS, _DIM), lambda i: (i, 0, 0, 0)),
            pl.BlockSpec((1, _BS, 1), lambda i: (i, 0, 0)),
            pl.BlockSpec((8, _DIM), lambda i: (0, 0)),
            pl.BlockSpec((_BS, _DIM), lambda i: (0, 0)),
        ],
        out_specs=pl.BlockSpec((_BS, _DIM), lambda i: (0, 0)),
        out_shape=jax.ShapeDtypeStruct((_BS, _DIM), jnp.int32),
    )(gum, u, wp, xi)
    return out.astype(xdtype)
